# baseline (speedup 1.0000x reference)
import jax
import jax.numpy as jnp
from jax import lax
from jax.experimental import pallas as pl
from jax.experimental.pallas import tpu as pltpu
from jax.experimental.pallas import tpu_sc as plsc

N = 50000
HID = 32
OUT = 4
G = 8
CONTENT_DIM = 310
BERT_DIM = 768

NC, NS = 2, 16
CHUNK = 128
K0 = 366
K1 = 30
KT = K0 + K1
GROUP = 3
E_PAD = NS * KT * CHUNK
RPT = 3126
N_PAD = NS * RPT
DUMP = 50000

R = 2000
GRID = N // R

_F32 = jnp.float32


def _sc_mesh():
    return plsc.VectorSubcoreMesh(core_axis_name="c", subcore_axis_name="s")


def _degree_sc(dstp, zeros16, ones16):
    def body(dst_hbm, z_hbm, ones_hbm, out_hbm, idx_d, ones_v, degtab, ssem):
        c = lax.axis_index("c")
        s = lax.axis_index("s")
        offs = jnp.where(c == 0, 0, K0)
        ng = jnp.where(c == 0, K0 // 6, K1 // 6)
        pltpu.sync_copy(dst_hbm.at[s], idx_d)
        pltpu.sync_copy(ones_hbm, ones_v)
        pltpu.sync_copy(z_hbm, degtab.at[pl.ds(s * RPT, RPT)])
        plsc.subcore_barrier()

        def grp(g, carry):
            for b in range(6):
                pltpu.async_copy(ones_v, degtab.at[idx_d.at[offs + g * 6 + b]], ssem, add=True)
            for b in range(6):
                pltpu.make_async_copy(ones_v, degtab.at[idx_d.at[offs + g * 6 + b]], ssem).wait()
            return carry

        lax.fori_loop(0, ng, grp, 0)
        plsc.subcore_barrier()
        pltpu.sync_copy(degtab.at[pl.ds(s * RPT, RPT)],
                        out_hbm.at[c, pl.ds(s * RPT, RPT)])

    return pl.kernel(
        body,
        out_type=jax.ShapeDtypeStruct((NC, N_PAD, 16), _F32),
        mesh=_sc_mesh(),
        scratch_types=[
            pltpu.VMEM((KT, CHUNK), jnp.int32),
            pltpu.VMEM((CHUNK, 16), _F32),
            pltpu.VMEM_SHARED((N_PAD, 16), _F32),
            pltpu.SemaphoreType.DMA,
        ],
        compiler_params=pltpu.CompilerParams(use_tc_tiling_on_sc=False),
    )(dstp, zeros16, ones16)


def _aggregate_sc(srcp, dstp, h, zeros32):
    def body(src_hbm, dst_hbm, h_hbm, z_hbm, out_hbm,
             ixs_a, ixd_a, ixs_b, ixd_b, rows, aggtab,
             isem_a, isem_b, gsem_a, gsem_b, ssem_a, ssem_b):
        c = lax.axis_index("c")
        s = lax.axis_index("s")
        offs = jnp.where(c == 0, 0, K0)
        ngroups = jnp.where(c == 0, K0 // GROUP, K1 // GROUP)
        pltpu.sync_copy(z_hbm, aggtab.at[pl.ds(s * RPT, RPT)])

        def load_idx(g, ixs, ixd, isem):
            pltpu.async_copy(src_hbm.at[s, pl.ds(offs + g * GROUP, GROUP)], ixs, isem)
            pltpu.async_copy(dst_hbm.at[s, pl.ds(offs + g * GROUP, GROUP)], ixd, isem)

        def wait_idx(g, ixs, ixd, isem):
            pltpu.make_async_copy(src_hbm.at[s, pl.ds(offs + g * GROUP, GROUP)], ixs, isem).wait()
            pltpu.make_async_copy(dst_hbm.at[s, pl.ds(offs + g * GROUP, GROUP)], ixd, isem).wait()

        def fire_gathers(ixs, base, gsem):
            for b in range(GROUP):
                pltpu.async_copy(h_hbm.at[ixs.at[b]], rows.at[base + b], gsem)

        def wait_gathers(ixs, base, gsem):
            for b in range(GROUP):
                pltpu.make_async_copy(h_hbm.at[ixs.at[b]], rows.at[base + b], gsem).wait()

        def fire_scatters(ixd, base, ssem):
            for b in range(GROUP):
                pltpu.async_copy(rows.at[base + b], aggtab.at[ixd.at[b]], ssem, add=True)

        def wait_scatters(ixd, base, ssem):
            for b in range(GROUP):
                pltpu.make_async_copy(rows.at[base + b], aggtab.at[ixd.at[b]], ssem).wait()

        load_idx(0, ixs_a, ixd_a, isem_a)
        wait_idx(0, ixs_a, ixd_a, isem_a)
        plsc.subcore_barrier()
        fire_gathers(ixs_a, 0, gsem_a)
        load_idx(1, ixs_b, ixd_b, isem_b)

        def step(gg, carry):
            g0 = 2 * gg
            g1 = g0 + 1
            wait_idx(g1, ixs_b, ixd_b, isem_b)
            fire_gathers(ixs_b, GROUP, gsem_b)
            wait_gathers(ixs_a, 0, gsem_a)
            fire_scatters(ixd_a, 0, ssem_a)
            wait_scatters(ixd_a, 0, ssem_a)

            @pl.when(g0 + 2 < ngroups)
            def _():
                load_idx(g0 + 2, ixs_a, ixd_a, isem_a)

            wait_gathers(ixs_b, GROUP, gsem_b)
            fire_scatters(ixd_b, GROUP, ssem_b)
            wait_scatters(ixd_b, GROUP, ssem_b)

            @pl.when(g0 + 2 < ngroups)
            def _():
                wait_idx(g0 + 2, ixs_a, ixd_a, isem_a)
                fire_gathers(ixs_a, 0, gsem_a)
                load_idx(g1 + 2, ixs_b, ixd_b, isem_b)

            return carry

        lax.fori_loop(0, ngroups // 2, step, 0)
        plsc.subcore_barrier()
        pltpu.sync_copy(aggtab.at[pl.ds(s * RPT, RPT)],
                        out_hbm.at[c, pl.ds(s * RPT, RPT)])

    return pl.kernel(
        body,
        out_type=jax.ShapeDtypeStruct((NC, N_PAD, HID), _F32),
        mesh=_sc_mesh(),
        scratch_types=[
            pltpu.VMEM((GROUP, CHUNK), jnp.int32),
            pltpu.VMEM((GROUP, CHUNK), jnp.int32),
            pltpu.VMEM((GROUP, CHUNK), jnp.int32),
            pltpu.VMEM((GROUP, CHUNK), jnp.int32),
            pltpu.VMEM((2 * GROUP, CHUNK, HID), _F32),
            pltpu.VMEM_SHARED((N_PAD, HID), _F32),
            pltpu.SemaphoreType.DMA,
            pltpu.SemaphoreType.DMA,
            pltpu.SemaphoreType.DMA,
            pltpu.SemaphoreType.DMA,
            pltpu.SemaphoreType.DMA,
            pltpu.SemaphoreType.DMA,
        ],
        compiler_params=pltpu.CompilerParams(use_tc_tiling_on_sc=False),
    )(srcp, dstp, h, zeros32)


def _encoder_tc(content_x, bert_x, degp, Wc, bc2, Wb, bb2, W1a, W1b):
    def body(cx, bx, dg, wc, bc, wb, bb, w1a, w1b, h1s_out, dinv_out):
        deg = dg[0, :, 0] + dg[1, :, 0] + 1.0
        dinv = lax.rsqrt(deg)[:, None]
        ch = jnp.maximum(jnp.dot(cx[...], wc[...], preferred_element_type=_F32) + bc[...], 0.0)
        bh = jnp.maximum(jnp.dot(bx[...], wb[...], preferred_element_type=_F32) + bb[...], 0.0)
        h1 = (jnp.dot(ch, w1a[...], preferred_element_type=_F32)
              + jnp.dot(bh, w1b[...], preferred_element_type=_F32))
        h1s_out[...] = h1 * dinv
        dinv_out[...] = dinv

    return pl.pallas_call(
        body,
        grid=(GRID,),
        in_specs=[
            pl.BlockSpec((R, CONTENT_DIM), lambda i: (i, 0)),
            pl.BlockSpec((R, BERT_DIM), lambda i: (i, 0)),
            pl.BlockSpec((NC, R, 16), lambda i: (0, i, 0)),
            pl.BlockSpec((CONTENT_DIM, HID), lambda i: (0, 0)),
            pl.BlockSpec((1, HID), lambda i: (0, 0)),
            pl.BlockSpec((BERT_DIM, HID), lambda i: (0, 0)),
            pl.BlockSpec((1, HID), lambda i: (0, 0)),
            pl.BlockSpec((HID, HID), lambda i: (0, 0)),
            pl.BlockSpec((HID, HID), lambda i: (0, 0)),
        ],
        out_specs=[
            pl.BlockSpec((R, HID), lambda i: (i, 0)),
            pl.BlockSpec((R, 1), lambda i: (i, 0)),
        ],
        out_shape=[
            jax.ShapeDtypeStruct((N, HID), _F32),
            jax.ShapeDtypeStruct((N, 1), _F32),
        ],
    )(content_x, bert_x, degp, Wc, bc2, Wb, bb2, W1a, W1b)


def _mid_tc(agg1, h1s, dinv, W2, b12):
    def body(ag, h1, dv, w2, b1, out):
        a = ag[0] + ag[1] + h1[...]
        o1 = jnp.maximum(a * dv[...] + b1[...], 0.0)
        out[...] = jnp.dot(o1, w2[...], preferred_element_type=_F32) * dv[...]

    return pl.pallas_call(
        body,
        grid=(GRID,),
        in_specs=[
            pl.BlockSpec((NC, R, HID), lambda i: (0, i, 0)),
            pl.BlockSpec((R, HID), lambda i: (i, 0)),
            pl.BlockSpec((R, 1), lambda i: (i, 0)),
            pl.BlockSpec((HID, HID), lambda i: (0, 0)),
            pl.BlockSpec((1, HID), lambda i: (0, 0)),
        ],
        out_specs=pl.BlockSpec((R, HID), lambda i: (i, 0)),
        out_shape=jax.ShapeDtypeStruct((N, HID), _F32),
    )(agg1, h1s, dinv, W2, b12)


def _final_tc(agg2, h2s, dinv, b22, batch2, Wl1, bl12, Wl2, bl22):
    def body(ag, h2, dv, b2, bt, wl1, bl1, wl2, bl2, out, acc, cnt):
        i = pl.program_id(0)
        a = ag[0] + ag[1] + h2[...]
        o2 = jnp.maximum(a * dv[...] + b2[...], 0.0)
        gids = lax.broadcasted_iota(jnp.int32, (R, G), 1)
        m = (bt[...] == gids).astype(_F32)
        psum = lax.dot_general(m, o2, (((0,), (0,)), ((), ())), preferred_element_type=_F32)
        pcnt = jnp.sum(m, axis=0)[:, None]

        @pl.when(i == 0)
        def _():
            acc[...] = jnp.zeros((G, HID), _F32)
            cnt[...] = jnp.zeros((G, HID), _F32)

        acc[...] = acc[...] + psum
        cnt[...] = cnt[...] + jnp.broadcast_to(pcnt, (G, HID))

        @pl.when(i == GRID - 1)
        def _():
            pooled = acc[...] / jnp.maximum(cnt[...], 1.0)
            z1 = jnp.maximum(jnp.dot(pooled, wl1[...], preferred_element_type=_F32) + bl1[...], 0.0)
            z = jnp.dot(z1, wl2[...], preferred_element_type=_F32) + bl2[...]
            zm = z - jnp.max(z, axis=-1, keepdims=True)
            out[...] = zm - jnp.log(jnp.sum(jnp.exp(zm), axis=-1, keepdims=True))

    return pl.pallas_call(
        body,
        grid=(GRID,),
        in_specs=[
            pl.BlockSpec((NC, R, HID), lambda i: (0, i, 0)),
            pl.BlockSpec((R, HID), lambda i: (i, 0)),
            pl.BlockSpec((R, 1), lambda i: (i, 0)),
            pl.BlockSpec((1, HID), lambda i: (0, 0)),
            pl.BlockSpec((R, 1), lambda i: (i, 0)),
            pl.BlockSpec((HID, HID), lambda i: (0, 0)),
            pl.BlockSpec((1, HID), lambda i: (0, 0)),
            pl.BlockSpec((HID, OUT), lambda i: (0, 0)),
            pl.BlockSpec((1, OUT), lambda i: (0, 0)),
        ],
        out_specs=pl.BlockSpec((G, OUT), lambda i: (0, 0)),
        out_shape=jax.ShapeDtypeStruct((G, OUT), _F32),
        scratch_shapes=[
            pltpu.VMEM((G, HID), _F32),
            pltpu.VMEM((G, HID), _F32),
        ],
    )(agg2, h2s, dinv, b22, batch2, Wl1, bl12, Wl2, bl22)


def kernel(content_x, bert_x, edge_index, batch, Wc, bc, Wb, bb,
           W1, b1, W2, b2, Wl1, bl1, Wl2, bl2):
    e = edge_index.shape[1]
    padn = E_PAD - e
    src = jnp.concatenate([edge_index[0], jnp.zeros((padn,), jnp.int32)])
    dst = jnp.concatenate([edge_index[1], jnp.full((padn,), DUMP, jnp.int32)])
    srcp = src.reshape(NS, KT, CHUNK)
    dstp = dst.reshape(NS, KT, CHUNK)

    zeros16 = jnp.zeros((RPT, 16), _F32)
    zeros32 = jnp.zeros((RPT, HID), _F32)
    ones16 = jnp.ones((CHUNK, 16), _F32)
    batch2 = batch.reshape(N, 1)
    bc2 = bc.reshape(1, HID)
    bb2 = bb.reshape(1, HID)
    b12 = b1.reshape(1, HID)
    b22 = b2.reshape(1, HID)
    bl12 = bl1.reshape(1, HID)
    bl22 = bl2.reshape(1, OUT)
    W1a = W1[:HID]
    W1b = W1[HID:]

    degp = _degree_sc(dstp, zeros16, ones16)
    h1s, dinv = _encoder_tc(content_x, bert_x, degp, Wc, bc2, Wb, bb2, W1a, W1b)
    agg1 = _aggregate_sc(srcp, dstp, h1s, zeros32)
    h2s = _mid_tc(agg1, h1s, dinv, W2, b12)
    agg2 = _aggregate_sc(srcp, dstp, h2s, zeros32)
    return _final_tc(agg2, h2s, dinv, b22, batch2, Wl1, bl12, Wl2, bl22)


# speedup vs baseline: 1.0952x; 1.0952x over previous
import jax
import jax.numpy as jnp
from jax import lax
from jax.experimental import pallas as pl
from jax.experimental.pallas import tpu as pltpu
from jax.experimental.pallas import tpu_sc as plsc

N = 50000
HID = 32
OUT = 4
G = 8
CONTENT_DIM = 310
BERT_DIM = 768

NC, NS = 2, 16
CHUNK = 128
K0 = 342
K1 = 54
KT = K0 + K1
GROUP = 3
E_PAD = NS * KT * CHUNK
RPT = 3126
N_PAD = NS * RPT
DUMP = 50000

R = 2000
GRID = N // R

_F32 = jnp.float32


def _sc_mesh():
    return plsc.VectorSubcoreMesh(core_axis_name="c", subcore_axis_name="s")


def _degree_sc(dstp, zeros16, ones16):
    def body(dst_hbm, z_hbm, ones_hbm, out_hbm, idx_d, ones_v, degtab, ssem):
        c = lax.axis_index("c")
        s = lax.axis_index("s")
        offs = jnp.where(c == 0, 0, K0)
        ng = jnp.where(c == 0, K0 // 6, K1 // 6)
        pltpu.sync_copy(dst_hbm.at[s], idx_d)
        pltpu.sync_copy(ones_hbm, ones_v)
        pltpu.sync_copy(z_hbm, degtab.at[pl.ds(s * RPT, RPT)])
        plsc.subcore_barrier()

        def grp(g, carry):
            for b in range(6):
                pltpu.async_copy(ones_v, degtab.at[idx_d.at[offs + g * 6 + b]], ssem, add=True)
            for b in range(6):
                pltpu.make_async_copy(ones_v, degtab.at[idx_d.at[offs + g * 6 + b]], ssem).wait()
            return carry

        lax.fori_loop(0, ng, grp, 0)
        plsc.subcore_barrier()
        pltpu.sync_copy(degtab.at[pl.ds(s * RPT, RPT)],
                        out_hbm.at[c, pl.ds(s * RPT, RPT)])

    return pl.kernel(
        body,
        out_type=jax.ShapeDtypeStruct((NC, N_PAD, 16), _F32),
        mesh=_sc_mesh(),
        scratch_types=[
            pltpu.VMEM((KT, CHUNK), jnp.int32),
            pltpu.VMEM((CHUNK, 16), _F32),
            pltpu.VMEM_SHARED((N_PAD, 16), _F32),
            pltpu.SemaphoreType.DMA,
        ],
        compiler_params=pltpu.CompilerParams(use_tc_tiling_on_sc=False),
    )(dstp, zeros16, ones16)


def _aggregate_sc(srcp, dstp, h, zeros32):
    def body(src_hbm, dst_hbm, h_hbm, z_hbm, out_hbm,
             ixs_a, ixd_a, ixs_b, ixd_b, rows, aggtab,
             isem_a, isem_b, gsem_a, gsem_b, ssem_a, ssem_b):
        c = lax.axis_index("c")
        s = lax.axis_index("s")
        offs = jnp.where(c == 0, 0, K0)
        ngroups = jnp.where(c == 0, K0 // GROUP, K1 // GROUP)
        pltpu.sync_copy(z_hbm, aggtab.at[pl.ds(s * RPT, RPT)])

        def load_idx(g, ixs, ixd, isem):
            pltpu.async_copy(src_hbm.at[s, pl.ds(offs + g * GROUP, GROUP)], ixs, isem)
            pltpu.async_copy(dst_hbm.at[s, pl.ds(offs + g * GROUP, GROUP)], ixd, isem)

        def wait_idx(g, ixs, ixd, isem):
            pltpu.make_async_copy(src_hbm.at[s, pl.ds(offs + g * GROUP, GROUP)], ixs, isem).wait()
            pltpu.make_async_copy(dst_hbm.at[s, pl.ds(offs + g * GROUP, GROUP)], ixd, isem).wait()

        def fire_gathers(ixs, base, gsem):
            for b in range(GROUP):
                pltpu.async_copy(h_hbm.at[ixs.at[b]], rows.at[base + b], gsem)

        def wait_gathers(ixs, base, gsem):
            for b in range(GROUP):
                pltpu.make_async_copy(h_hbm.at[ixs.at[b]], rows.at[base + b], gsem).wait()

        def fire_scatters(ixd, base, ssem):
            for b in range(GROUP):
                pltpu.async_copy(rows.at[base + b], aggtab.at[ixd.at[b]], ssem, add=True)

        def wait_scatters(ixd, base, ssem):
            for b in range(GROUP):
                pltpu.make_async_copy(rows.at[base + b], aggtab.at[ixd.at[b]], ssem).wait()

        load_idx(0, ixs_a, ixd_a, isem_a)
        wait_idx(0, ixs_a, ixd_a, isem_a)
        plsc.subcore_barrier()
        fire_gathers(ixs_a, 0, gsem_a)
        load_idx(1, ixs_b, ixd_b, isem_b)

        def step(gg, carry):
            g0 = 2 * gg
            g1 = g0 + 1
            wait_idx(g1, ixs_b, ixd_b, isem_b)
            fire_gathers(ixs_b, GROUP, gsem_b)
            wait_gathers(ixs_a, 0, gsem_a)
            fire_scatters(ixd_a, 0, ssem_a)
            wait_scatters(ixd_a, 0, ssem_a)

            @pl.when(g0 + 2 < ngroups)
            def _():
                load_idx(g0 + 2, ixs_a, ixd_a, isem_a)

            wait_gathers(ixs_b, GROUP, gsem_b)
            fire_scatters(ixd_b, GROUP, ssem_b)
            wait_scatters(ixd_b, GROUP, ssem_b)

            @pl.when(g0 + 2 < ngroups)
            def _():
                wait_idx(g0 + 2, ixs_a, ixd_a, isem_a)
                fire_gathers(ixs_a, 0, gsem_a)
                load_idx(g1 + 2, ixs_b, ixd_b, isem_b)

            return carry

        lax.fori_loop(0, ngroups // 2, step, 0)
        plsc.subcore_barrier()
        pltpu.sync_copy(aggtab.at[pl.ds(s * RPT, RPT)],
                        out_hbm.at[c, pl.ds(s * RPT, RPT)])

    return pl.kernel(
        body,
        out_type=jax.ShapeDtypeStruct((NC, N_PAD, HID), _F32),
        mesh=_sc_mesh(),
        scratch_types=[
            pltpu.VMEM((GROUP, CHUNK), jnp.int32),
            pltpu.VMEM((GROUP, CHUNK), jnp.int32),
            pltpu.VMEM((GROUP, CHUNK), jnp.int32),
            pltpu.VMEM((GROUP, CHUNK), jnp.int32),
            pltpu.VMEM((2 * GROUP, CHUNK, HID), _F32),
            pltpu.VMEM_SHARED((N_PAD, HID), _F32),
            pltpu.SemaphoreType.DMA,
            pltpu.SemaphoreType.DMA,
            pltpu.SemaphoreType.DMA,
            pltpu.SemaphoreType.DMA,
            pltpu.SemaphoreType.DMA,
            pltpu.SemaphoreType.DMA,
        ],
        compiler_params=pltpu.CompilerParams(use_tc_tiling_on_sc=False),
    )(srcp, dstp, h, zeros32)


def _encoder_tc(content_x, bert_x, degp, Wc, bc2, Wb, bb2, W1a, W1b):
    def body(cx, bx, dg, wc, bc, wb, bb, w1a, w1b, h1s_out, dinv_out):
        deg = dg[0, :, 0] + dg[1, :, 0] + 1.0
        dinv = lax.rsqrt(deg)[:, None]
        ch = jnp.maximum(jnp.dot(cx[...], wc[...], preferred_element_type=_F32) + bc[...], 0.0)
        bh = jnp.maximum(jnp.dot(bx[...], wb[...], preferred_element_type=_F32) + bb[...], 0.0)
        h1 = (jnp.dot(ch, w1a[...], preferred_element_type=_F32)
              + jnp.dot(bh, w1b[...], preferred_element_type=_F32))
        h1s_out[...] = h1 * dinv
        dinv_out[...] = dinv

    return pl.pallas_call(
        body,
        grid=(GRID,),
        in_specs=[
            pl.BlockSpec((R, CONTENT_DIM), lambda i: (i, 0)),
            pl.BlockSpec((R, BERT_DIM), lambda i: (i, 0)),
            pl.BlockSpec((NC, R, 16), lambda i: (0, i, 0)),
            pl.BlockSpec((CONTENT_DIM, HID), lambda i: (0, 0)),
            pl.BlockSpec((1, HID), lambda i: (0, 0)),
            pl.BlockSpec((BERT_DIM, HID), lambda i: (0, 0)),
            pl.BlockSpec((1, HID), lambda i: (0, 0)),
            pl.BlockSpec((HID, HID), lambda i: (0, 0)),
            pl.BlockSpec((HID, HID), lambda i: (0, 0)),
        ],
        out_specs=[
            pl.BlockSpec((R, HID), lambda i: (i, 0)),
            pl.BlockSpec((R, 1), lambda i: (i, 0)),
        ],
        out_shape=[
            jax.ShapeDtypeStruct((N, HID), _F32),
            jax.ShapeDtypeStruct((N, 1), _F32),
        ],
    )(content_x, bert_x, degp, Wc, bc2, Wb, bb2, W1a, W1b)


def _mid_tc(agg1, h1s, dinv, W2, b12):
    def body(ag, h1, dv, w2, b1, out):
        a = ag[0] + ag[1] + h1[...]
        o1 = jnp.maximum(a * dv[...] + b1[...], 0.0)
        out[...] = jnp.dot(o1, w2[...], preferred_element_type=_F32) * dv[...]

    return pl.pallas_call(
        body,
        grid=(GRID,),
        in_specs=[
            pl.BlockSpec((NC, R, HID), lambda i: (0, i, 0)),
            pl.BlockSpec((R, HID), lambda i: (i, 0)),
            pl.BlockSpec((R, 1), lambda i: (i, 0)),
            pl.BlockSpec((HID, HID), lambda i: (0, 0)),
            pl.BlockSpec((1, HID), lambda i: (0, 0)),
        ],
        out_specs=pl.BlockSpec((R, HID), lambda i: (i, 0)),
        out_shape=jax.ShapeDtypeStruct((N, HID), _F32),
    )(agg1, h1s, dinv, W2, b12)


def _final_tc(agg2, h2s, dinv, b22, batch2, Wl1, bl12, Wl2, bl22):
    def body(ag, h2, dv, b2, bt, wl1, bl1, wl2, bl2, out, acc, cnt):
        i = pl.program_id(0)
        a = ag[0] + ag[1] + h2[...]
        o2 = jnp.maximum(a * dv[...] + b2[...], 0.0)
        gids = lax.broadcasted_iota(jnp.int32, (R, G), 1)
        m = (bt[...] == gids).astype(_F32)
        psum = lax.dot_general(m, o2, (((0,), (0,)), ((), ())), preferred_element_type=_F32)
        pcnt = jnp.sum(m, axis=0)[:, None]

        @pl.when(i == 0)
        def _():
            acc[...] = jnp.zeros((G, HID), _F32)
            cnt[...] = jnp.zeros((G, HID), _F32)

        acc[...] = acc[...] + psum
        cnt[...] = cnt[...] + jnp.broadcast_to(pcnt, (G, HID))

        @pl.when(i == GRID - 1)
        def _():
            pooled = acc[...] / jnp.maximum(cnt[...], 1.0)
            z1 = jnp.maximum(jnp.dot(pooled, wl1[...], preferred_element_type=_F32) + bl1[...], 0.0)
            z = jnp.dot(z1, wl2[...], preferred_element_type=_F32) + bl2[...]
            zm = z - jnp.max(z, axis=-1, keepdims=True)
            out[...] = zm - jnp.log(jnp.sum(jnp.exp(zm), axis=-1, keepdims=True))

    return pl.pallas_call(
        body,
        grid=(GRID,),
        in_specs=[
            pl.BlockSpec((NC, R, HID), lambda i: (0, i, 0)),
            pl.BlockSpec((R, HID), lambda i: (i, 0)),
            pl.BlockSpec((R, 1), lambda i: (i, 0)),
            pl.BlockSpec((1, HID), lambda i: (0, 0)),
            pl.BlockSpec((R, 1), lambda i: (i, 0)),
            pl.BlockSpec((HID, HID), lambda i: (0, 0)),
            pl.BlockSpec((1, HID), lambda i: (0, 0)),
            pl.BlockSpec((HID, OUT), lambda i: (0, 0)),
            pl.BlockSpec((1, OUT), lambda i: (0, 0)),
        ],
        out_specs=pl.BlockSpec((G, OUT), lambda i: (0, 0)),
        out_shape=jax.ShapeDtypeStruct((G, OUT), _F32),
        scratch_shapes=[
            pltpu.VMEM((G, HID), _F32),
            pltpu.VMEM((G, HID), _F32),
        ],
    )(agg2, h2s, dinv, b22, batch2, Wl1, bl12, Wl2, bl22)


def kernel(content_x, bert_x, edge_index, batch, Wc, bc, Wb, bb,
           W1, b1, W2, b2, Wl1, bl1, Wl2, bl2):
    e = edge_index.shape[1]
    padn = E_PAD - e
    src = jnp.concatenate([edge_index[0], jnp.zeros((padn,), jnp.int32)])
    dst = jnp.concatenate([edge_index[1], jnp.full((padn,), DUMP, jnp.int32)])
    srcp = src.reshape(NS, KT, CHUNK)
    dstp = dst.reshape(NS, KT, CHUNK)

    zeros16 = jnp.zeros((RPT, 16), _F32)
    zeros32 = jnp.zeros((RPT, HID), _F32)
    ones16 = jnp.ones((CHUNK, 16), _F32)
    batch2 = batch.reshape(N, 1)
    bc2 = bc.reshape(1, HID)
    bb2 = bb.reshape(1, HID)
    b12 = b1.reshape(1, HID)
    b22 = b2.reshape(1, HID)
    bl12 = bl1.reshape(1, HID)
    bl22 = bl2.reshape(1, OUT)
    W1a = W1[:HID]
    W1b = W1[HID:]

    degp = _degree_sc(dstp, zeros16, ones16)
    h1s, dinv = _encoder_tc(content_x, bert_x, degp, Wc, bc2, Wb, bb2, W1a, W1b)
    agg1 = _aggregate_sc(srcp, dstp, h1s, zeros32)
    h2s = _mid_tc(agg1, h1s, dinv, W2, b12)
    agg2 = _aggregate_sc(srcp, dstp, h2s, zeros32)
    return _final_tc(agg2, h2s, dinv, b22, batch2, Wl1, bl12, Wl2, bl22)


# speedup vs baseline: 1.1332x; 1.0347x over previous
import jax
import jax.numpy as jnp
from jax import lax
from jax.experimental import pallas as pl
from jax.experimental.pallas import tpu as pltpu
from jax.experimental.pallas import tpu_sc as plsc

N = 50000
HID = 32
OUT = 4
G = 8
CONTENT_DIM = 310
BERT_DIM = 768

NC, NS = 2, 16
CHUNK = 128
K0 = 330
K1 = 66
KT = K0 + K1
GROUP = 3
E_PAD = NS * KT * CHUNK
RPT = 3126
N_PAD = NS * RPT
DUMP = 50000

R = 2000
GRID = N // R

_F32 = jnp.float32


def _sc_mesh():
    return plsc.VectorSubcoreMesh(core_axis_name="c", subcore_axis_name="s")


def _degree_sc(dstp, zeros16, ones16):
    def body(dst_hbm, z_hbm, ones_hbm, out_hbm, idx_d, ones_v, degtab, ssem):
        c = lax.axis_index("c")
        s = lax.axis_index("s")
        offs = jnp.where(c == 0, 0, K0)
        ng = jnp.where(c == 0, K0 // 6, K1 // 6)
        pltpu.sync_copy(dst_hbm.at[s], idx_d)
        pltpu.sync_copy(ones_hbm, ones_v)
        pltpu.sync_copy(z_hbm, degtab.at[pl.ds(s * RPT, RPT)])
        plsc.subcore_barrier()

        def grp(g, carry):
            for b in range(6):
                pltpu.async_copy(ones_v, degtab.at[idx_d.at[offs + g * 6 + b]], ssem, add=True)
            for b in range(6):
                pltpu.make_async_copy(ones_v, degtab.at[idx_d.at[offs + g * 6 + b]], ssem).wait()
            return carry

        lax.fori_loop(0, ng, grp, 0)
        plsc.subcore_barrier()
        pltpu.sync_copy(degtab.at[pl.ds(s * RPT, RPT)],
                        out_hbm.at[c, pl.ds(s * RPT, RPT)])

    return pl.kernel(
        body,
        out_type=jax.ShapeDtypeStruct((NC, N_PAD, 16), _F32),
        mesh=_sc_mesh(),
        scratch_types=[
            pltpu.VMEM((KT, CHUNK), jnp.int32),
            pltpu.VMEM((CHUNK, 16), _F32),
            pltpu.VMEM_SHARED((N_PAD, 16), _F32),
            pltpu.SemaphoreType.DMA,
        ],
        compiler_params=pltpu.CompilerParams(use_tc_tiling_on_sc=False),
    )(dstp, zeros16, ones16)


def _aggregate_sc(srcp, dstp, h, zeros32):
    def body(src_hbm, dst_hbm, h_hbm, z_hbm, out_hbm,
             ixs_a, ixd_a, ixs_b, ixd_b, rows, aggtab,
             isem_a, isem_b, gsem_a, gsem_b, ssem_a, ssem_b):
        c = lax.axis_index("c")
        s = lax.axis_index("s")
        offs = jnp.where(c == 0, 0, K0)
        ngroups = jnp.where(c == 0, K0 // GROUP, K1 // GROUP)
        pltpu.sync_copy(z_hbm, aggtab.at[pl.ds(s * RPT, RPT)])

        def load_idx(g, ixs, ixd, isem):
            pltpu.async_copy(src_hbm.at[s, pl.ds(offs + g * GROUP, GROUP)], ixs, isem)
            pltpu.async_copy(dst_hbm.at[s, pl.ds(offs + g * GROUP, GROUP)], ixd, isem)

        def wait_idx(g, ixs, ixd, isem):
            pltpu.make_async_copy(src_hbm.at[s, pl.ds(offs + g * GROUP, GROUP)], ixs, isem).wait()
            pltpu.make_async_copy(dst_hbm.at[s, pl.ds(offs + g * GROUP, GROUP)], ixd, isem).wait()

        def fire_gathers(ixs, base, gsem):
            for b in range(GROUP):
                pltpu.async_copy(h_hbm.at[ixs.at[b]], rows.at[base + b], gsem)

        def wait_gathers(ixs, base, gsem):
            for b in range(GROUP):
                pltpu.make_async_copy(h_hbm.at[ixs.at[b]], rows.at[base + b], gsem).wait()

        def fire_scatters(ixd, base, ssem):
            for b in range(GROUP):
                pltpu.async_copy(rows.at[base + b], aggtab.at[ixd.at[b]], ssem, add=True)

        def wait_scatters(ixd, base, ssem):
            for b in range(GROUP):
                pltpu.make_async_copy(rows.at[base + b], aggtab.at[ixd.at[b]], ssem).wait()

        load_idx(0, ixs_a, ixd_a, isem_a)
        wait_idx(0, ixs_a, ixd_a, isem_a)
        plsc.subcore_barrier()
        fire_gathers(ixs_a, 0, gsem_a)
        load_idx(1, ixs_b, ixd_b, isem_b)

        def step(gg, carry):
            g0 = 2 * gg
            g1 = g0 + 1
            wait_idx(g1, ixs_b, ixd_b, isem_b)
            fire_gathers(ixs_b, GROUP, gsem_b)
            wait_gathers(ixs_a, 0, gsem_a)
            fire_scatters(ixd_a, 0, ssem_a)
            wait_gathers(ixs_b, GROUP, gsem_b)
            fire_scatters(ixd_b, GROUP, ssem_b)
            wait_scatters(ixd_a, 0, ssem_a)

            @pl.when(g0 + 2 < ngroups)
            def _():
                load_idx(g0 + 2, ixs_a, ixd_a, isem_a)
                wait_idx(g0 + 2, ixs_a, ixd_a, isem_a)
                fire_gathers(ixs_a, 0, gsem_a)

            wait_scatters(ixd_b, GROUP, ssem_b)

            @pl.when(g0 + 2 < ngroups)
            def _():
                load_idx(g1 + 2, ixs_b, ixd_b, isem_b)

            return carry

        lax.fori_loop(0, ngroups // 2, step, 0)
        plsc.subcore_barrier()
        pltpu.sync_copy(aggtab.at[pl.ds(s * RPT, RPT)],
                        out_hbm.at[c, pl.ds(s * RPT, RPT)])

    return pl.kernel(
        body,
        out_type=jax.ShapeDtypeStruct((NC, N_PAD, HID), _F32),
        mesh=_sc_mesh(),
        scratch_types=[
            pltpu.VMEM((GROUP, CHUNK), jnp.int32),
            pltpu.VMEM((GROUP, CHUNK), jnp.int32),
            pltpu.VMEM((GROUP, CHUNK), jnp.int32),
            pltpu.VMEM((GROUP, CHUNK), jnp.int32),
            pltpu.VMEM((2 * GROUP, CHUNK, HID), _F32),
            pltpu.VMEM_SHARED((N_PAD, HID), _F32),
            pltpu.SemaphoreType.DMA,
            pltpu.SemaphoreType.DMA,
            pltpu.SemaphoreType.DMA,
            pltpu.SemaphoreType.DMA,
            pltpu.SemaphoreType.DMA,
            pltpu.SemaphoreType.DMA,
        ],
        compiler_params=pltpu.CompilerParams(use_tc_tiling_on_sc=False),
    )(srcp, dstp, h, zeros32)


def _encoder_tc(content_x, bert_x, degp, Wc, bc2, Wb, bb2, W1a, W1b):
    def body(cx, bx, dg, wc, bc, wb, bb, w1a, w1b, h1s_out, dinv_out):
        deg = dg[0, :, 0] + dg[1, :, 0] + 1.0
        dinv = lax.rsqrt(deg)[:, None]
        ch = jnp.maximum(jnp.dot(cx[...], wc[...], preferred_element_type=_F32) + bc[...], 0.0)
        bh = jnp.maximum(jnp.dot(bx[...], wb[...], preferred_element_type=_F32) + bb[...], 0.0)
        h1 = (jnp.dot(ch, w1a[...], preferred_element_type=_F32)
              + jnp.dot(bh, w1b[...], preferred_element_type=_F32))
        h1s_out[...] = h1 * dinv
        dinv_out[...] = dinv

    return pl.pallas_call(
        body,
        grid=(GRID,),
        in_specs=[
            pl.BlockSpec((R, CONTENT_DIM), lambda i: (i, 0)),
            pl.BlockSpec((R, BERT_DIM), lambda i: (i, 0)),
            pl.BlockSpec((NC, R, 16), lambda i: (0, i, 0)),
            pl.BlockSpec((CONTENT_DIM, HID), lambda i: (0, 0)),
            pl.BlockSpec((1, HID), lambda i: (0, 0)),
            pl.BlockSpec((BERT_DIM, HID), lambda i: (0, 0)),
            pl.BlockSpec((1, HID), lambda i: (0, 0)),
            pl.BlockSpec((HID, HID), lambda i: (0, 0)),
            pl.BlockSpec((HID, HID), lambda i: (0, 0)),
        ],
        out_specs=[
            pl.BlockSpec((R, HID), lambda i: (i, 0)),
            pl.BlockSpec((R, 1), lambda i: (i, 0)),
        ],
        out_shape=[
            jax.ShapeDtypeStruct((N, HID), _F32),
            jax.ShapeDtypeStruct((N, 1), _F32),
        ],
    )(content_x, bert_x, degp, Wc, bc2, Wb, bb2, W1a, W1b)


def _mid_tc(agg1, h1s, dinv, W2, b12):
    def body(ag, h1, dv, w2, b1, out):
        a = ag[0] + ag[1] + h1[...]
        o1 = jnp.maximum(a * dv[...] + b1[...], 0.0)
        out[...] = jnp.dot(o1, w2[...], preferred_element_type=_F32) * dv[...]

    return pl.pallas_call(
        body,
        grid=(GRID,),
        in_specs=[
            pl.BlockSpec((NC, R, HID), lambda i: (0, i, 0)),
            pl.BlockSpec((R, HID), lambda i: (i, 0)),
            pl.BlockSpec((R, 1), lambda i: (i, 0)),
            pl.BlockSpec((HID, HID), lambda i: (0, 0)),
            pl.BlockSpec((1, HID), lambda i: (0, 0)),
        ],
        out_specs=pl.BlockSpec((R, HID), lambda i: (i, 0)),
        out_shape=jax.ShapeDtypeStruct((N, HID), _F32),
    )(agg1, h1s, dinv, W2, b12)


def _final_tc(agg2, h2s, dinv, b22, batch2, Wl1, bl12, Wl2, bl22):
    def body(ag, h2, dv, b2, bt, wl1, bl1, wl2, bl2, out, acc, cnt):
        i = pl.program_id(0)
        a = ag[0] + ag[1] + h2[...]
        o2 = jnp.maximum(a * dv[...] + b2[...], 0.0)
        gids = lax.broadcasted_iota(jnp.int32, (R, G), 1)
        m = (bt[...] == gids).astype(_F32)
        psum = lax.dot_general(m, o2, (((0,), (0,)), ((), ())), preferred_element_type=_F32)
        pcnt = jnp.sum(m, axis=0)[:, None]

        @pl.when(i == 0)
        def _():
            acc[...] = jnp.zeros((G, HID), _F32)
            cnt[...] = jnp.zeros((G, HID), _F32)

        acc[...] = acc[...] + psum
        cnt[...] = cnt[...] + jnp.broadcast_to(pcnt, (G, HID))

        @pl.when(i == GRID - 1)
        def _():
            pooled = acc[...] / jnp.maximum(cnt[...], 1.0)
            z1 = jnp.maximum(jnp.dot(pooled, wl1[...], preferred_element_type=_F32) + bl1[...], 0.0)
            z = jnp.dot(z1, wl2[...], preferred_element_type=_F32) + bl2[...]
            zm = z - jnp.max(z, axis=-1, keepdims=True)
            out[...] = zm - jnp.log(jnp.sum(jnp.exp(zm), axis=-1, keepdims=True))

    return pl.pallas_call(
        body,
        grid=(GRID,),
        in_specs=[
            pl.BlockSpec((NC, R, HID), lambda i: (0, i, 0)),
            pl.BlockSpec((R, HID), lambda i: (i, 0)),
            pl.BlockSpec((R, 1), lambda i: (i, 0)),
            pl.BlockSpec((1, HID), lambda i: (0, 0)),
            pl.BlockSpec((R, 1), lambda i: (i, 0)),
            pl.BlockSpec((HID, HID), lambda i: (0, 0)),
            pl.BlockSpec((1, HID), lambda i: (0, 0)),
            pl.BlockSpec((HID, OUT), lambda i: (0, 0)),
            pl.BlockSpec((1, OUT), lambda i: (0, 0)),
        ],
        out_specs=pl.BlockSpec((G, OUT), lambda i: (0, 0)),
        out_shape=jax.ShapeDtypeStruct((G, OUT), _F32),
        scratch_shapes=[
            pltpu.VMEM((G, HID), _F32),
            pltpu.VMEM((G, HID), _F32),
        ],
    )(agg2, h2s, dinv, b22, batch2, Wl1, bl12, Wl2, bl22)


def kernel(content_x, bert_x, edge_index, batch, Wc, bc, Wb, bb,
           W1, b1, W2, b2, Wl1, bl1, Wl2, bl2):
    e = edge_index.shape[1]
    padn = E_PAD - e
    src = jnp.concatenate([edge_index[0], jnp.zeros((padn,), jnp.int32)])
    dst = jnp.concatenate([edge_index[1], jnp.full((padn,), DUMP, jnp.int32)])
    srcp = src.reshape(NS, KT, CHUNK)
    dstp = dst.reshape(NS, KT, CHUNK)

    zeros16 = jnp.zeros((RPT, 16), _F32)
    zeros32 = jnp.zeros((RPT, HID), _F32)
    ones16 = jnp.ones((CHUNK, 16), _F32)
    batch2 = batch.reshape(N, 1)
    bc2 = bc.reshape(1, HID)
    bb2 = bb.reshape(1, HID)
    b12 = b1.reshape(1, HID)
    b22 = b2.reshape(1, HID)
    bl12 = bl1.reshape(1, HID)
    bl22 = bl2.reshape(1, OUT)
    W1a = W1[:HID]
    W1b = W1[HID:]

    degp = _degree_sc(dstp, zeros16, ones16)
    h1s, dinv = _encoder_tc(content_x, bert_x, degp, Wc, bc2, Wb, bb2, W1a, W1b)
    agg1 = _aggregate_sc(srcp, dstp, h1s, zeros32)
    h2s = _mid_tc(agg1, h1s, dinv, W2, b12)
    agg2 = _aggregate_sc(srcp, dstp, h2s, zeros32)
    return _final_tc(agg2, h2s, dinv, b22, batch2, Wl1, bl12, Wl2, bl22)


# speedup vs baseline: 1.1437x; 1.0093x over previous
import jax
import jax.numpy as jnp
from jax import lax
from jax.experimental import pallas as pl
from jax.experimental.pallas import tpu as pltpu
from jax.experimental.pallas import tpu_sc as plsc

N = 50000
HID = 32
OUT = 4
G = 8
CONTENT_DIM = 310
BERT_DIM = 768

NC, NS = 2, 16
CHUNK = 128
K0 = 330
K1 = 66
KT = K0 + K1
GROUP = 3
E_PAD = NS * KT * CHUNK
RPT = 3126
N_PAD = NS * RPT
DUMP = 50000

R = 2000
GRID = N // R

_F32 = jnp.float32


def _sc_mesh():
    return plsc.VectorSubcoreMesh(core_axis_name="c", subcore_axis_name="s")


def _degree_sc(dstp, zeros16, ones16):
    def body(dst_hbm, z_hbm, ones_hbm, out_hbm, idx_d, ones_v, degtab, ssem):
        c = lax.axis_index("c")
        s = lax.axis_index("s")
        offs = jnp.where(c == 0, 0, K0)
        ng = jnp.where(c == 0, K0 // 6, K1 // 6)
        pltpu.sync_copy(dst_hbm.at[s], idx_d)
        pltpu.sync_copy(ones_hbm, ones_v)
        pltpu.sync_copy(z_hbm, degtab.at[pl.ds(s * RPT, RPT)])
        plsc.subcore_barrier()

        def grp(g, carry):
            for b in range(6):
                pltpu.async_copy(ones_v, degtab.at[idx_d.at[offs + g * 6 + b]], ssem, add=True)
            for b in range(6):
                pltpu.make_async_copy(ones_v, degtab.at[idx_d.at[offs + g * 6 + b]], ssem).wait()
            return carry

        lax.fori_loop(0, ng, grp, 0)
        plsc.subcore_barrier()
        pltpu.sync_copy(degtab.at[pl.ds(s * RPT, RPT)],
                        out_hbm.at[c, pl.ds(s * RPT, RPT)])

    return pl.kernel(
        body,
        out_type=jax.ShapeDtypeStruct((NC, N_PAD, 16), _F32),
        mesh=_sc_mesh(),
        scratch_types=[
            pltpu.VMEM((KT, CHUNK), jnp.int32),
            pltpu.VMEM((CHUNK, 16), _F32),
            pltpu.VMEM_SHARED((N_PAD, 16), _F32),
            pltpu.SemaphoreType.DMA,
        ],
        compiler_params=pltpu.CompilerParams(use_tc_tiling_on_sc=False),
    )(dstp, zeros16, ones16)


def _aggregate_sc(srcp, dstp, h, zeros32):
    def body(src_hbm, dst_hbm, h_hbm, z_hbm, out_hbm,
             ixs_a, ixd_a, ixs_b, ixd_b, rows, aggtab,
             isem_a, isem_b, gsem_a, gsem_b, ssem_a, ssem_b):
        c = lax.axis_index("c")
        s = lax.axis_index("s")
        offs = jnp.where(c == 0, 0, K0)
        ngroups = jnp.where(c == 0, K0 // GROUP, K1 // GROUP)
        pltpu.sync_copy(z_hbm, aggtab.at[pl.ds(s * RPT, RPT)])

        def load_idx(g, ixs, ixd, isem):
            pltpu.async_copy(src_hbm.at[s, pl.ds(offs + g * GROUP, GROUP)], ixs, isem)
            pltpu.async_copy(dst_hbm.at[s, pl.ds(offs + g * GROUP, GROUP)], ixd, isem)

        def wait_idx(g, ixs, ixd, isem):
            pltpu.make_async_copy(src_hbm.at[s, pl.ds(offs + g * GROUP, GROUP)], ixs, isem).wait()
            pltpu.make_async_copy(dst_hbm.at[s, pl.ds(offs + g * GROUP, GROUP)], ixd, isem).wait()

        def fire_gathers(ixs, base, gsem):
            for b in range(GROUP):
                pltpu.async_copy(h_hbm.at[ixs.at[b]], rows.at[base + b], gsem)

        def wait_gathers(ixs, base, gsem):
            for b in range(GROUP):
                pltpu.make_async_copy(h_hbm.at[ixs.at[b]], rows.at[base + b], gsem).wait()

        def fire_scatters(ixd, base, ssem):
            for b in range(GROUP):
                pltpu.async_copy(rows.at[base + b], aggtab.at[ixd.at[b]], ssem, add=True)

        def wait_scatters(ixd, base, ssem):
            for b in range(GROUP):
                pltpu.make_async_copy(rows.at[base + b], aggtab.at[ixd.at[b]], ssem).wait()

        load_idx(0, ixs_a, ixd_a, isem_a)
        wait_idx(0, ixs_a, ixd_a, isem_a)
        plsc.subcore_barrier()
        fire_gathers(ixs_a, 0, gsem_a)
        load_idx(1, ixs_b, ixd_b, isem_b)

        def step(gg, carry):
            g0 = 2 * gg
            g1 = g0 + 1
            wait_idx(g1, ixs_b, ixd_b, isem_b)
            fire_gathers(ixs_b, GROUP, gsem_b)
            wait_gathers(ixs_a, 0, gsem_a)
            fire_scatters(ixd_a, 0, ssem_a)
            wait_scatters(ixd_a, 0, ssem_a)

            @pl.when(g0 + 2 < ngroups)
            def _():
                load_idx(g0 + 2, ixs_a, ixd_a, isem_a)

            wait_gathers(ixs_b, GROUP, gsem_b)
            fire_scatters(ixd_b, GROUP, ssem_b)
            wait_scatters(ixd_b, GROUP, ssem_b)

            @pl.when(g0 + 2 < ngroups)
            def _():
                wait_idx(g0 + 2, ixs_a, ixd_a, isem_a)
                fire_gathers(ixs_a, 0, gsem_a)
                load_idx(g1 + 2, ixs_b, ixd_b, isem_b)

            return carry

        lax.fori_loop(0, ngroups // 2, step, 0)
        plsc.subcore_barrier()
        pltpu.sync_copy(aggtab.at[pl.ds(s * RPT, RPT)],
                        out_hbm.at[c, pl.ds(s * RPT, RPT)])

    return pl.kernel(
        body,
        out_type=jax.ShapeDtypeStruct((NC, N_PAD, HID), _F32),
        mesh=_sc_mesh(),
        scratch_types=[
            pltpu.VMEM((GROUP, CHUNK), jnp.int32),
            pltpu.VMEM((GROUP, CHUNK), jnp.int32),
            pltpu.VMEM((GROUP, CHUNK), jnp.int32),
            pltpu.VMEM((GROUP, CHUNK), jnp.int32),
            pltpu.VMEM((2 * GROUP, CHUNK, HID), _F32),
            pltpu.VMEM_SHARED((N_PAD, HID), _F32),
            pltpu.SemaphoreType.DMA,
            pltpu.SemaphoreType.DMA,
            pltpu.SemaphoreType.DMA,
            pltpu.SemaphoreType.DMA,
            pltpu.SemaphoreType.DMA,
            pltpu.SemaphoreType.DMA,
        ],
        compiler_params=pltpu.CompilerParams(use_tc_tiling_on_sc=False),
    )(srcp, dstp, h, zeros32)


def _encoder_tc(content_x, bert_x, degp, Wc, bc2, Wb, bb2, W1a, W1b):
    def body(cx, bx, dg, wc, bc, wb, bb, w1a, w1b, h1s_out, dinv_out):
        deg = dg[0, :, 0] + dg[1, :, 0] + 1.0
        dinv = lax.rsqrt(deg)[:, None]
        ch = jnp.maximum(jnp.dot(cx[...], wc[...], preferred_element_type=_F32) + bc[...], 0.0)
        bh = jnp.maximum(jnp.dot(bx[...], wb[...], preferred_element_type=_F32) + bb[...], 0.0)
        h1 = (jnp.dot(ch, w1a[...], preferred_element_type=_F32)
              + jnp.dot(bh, w1b[...], preferred_element_type=_F32))
        h1s_out[...] = h1 * dinv
        dinv_out[...] = dinv

    return pl.pallas_call(
        body,
        grid=(GRID,),
        in_specs=[
            pl.BlockSpec((R, CONTENT_DIM), lambda i: (i, 0)),
            pl.BlockSpec((R, BERT_DIM), lambda i: (i, 0)),
            pl.BlockSpec((NC, R, 16), lambda i: (0, i, 0)),
            pl.BlockSpec((CONTENT_DIM, HID), lambda i: (0, 0)),
            pl.BlockSpec((1, HID), lambda i: (0, 0)),
            pl.BlockSpec((BERT_DIM, HID), lambda i: (0, 0)),
            pl.BlockSpec((1, HID), lambda i: (0, 0)),
            pl.BlockSpec((HID, HID), lambda i: (0, 0)),
            pl.BlockSpec((HID, HID), lambda i: (0, 0)),
        ],
        out_specs=[
            pl.BlockSpec((R, HID), lambda i: (i, 0)),
            pl.BlockSpec((R, 1), lambda i: (i, 0)),
        ],
        out_shape=[
            jax.ShapeDtypeStruct((N, HID), _F32),
            jax.ShapeDtypeStruct((N, 1), _F32),
        ],
    )(content_x, bert_x, degp, Wc, bc2, Wb, bb2, W1a, W1b)


def _mid_tc(agg1, h1s, dinv, W2, b12):
    def body(ag, h1, dv, w2, b1, out):
        a = ag[0] + ag[1] + h1[...]
        o1 = jnp.maximum(a * dv[...] + b1[...], 0.0)
        out[...] = jnp.dot(o1, w2[...], preferred_element_type=_F32) * dv[...]

    return pl.pallas_call(
        body,
        grid=(GRID,),
        in_specs=[
            pl.BlockSpec((NC, R, HID), lambda i: (0, i, 0)),
            pl.BlockSpec((R, HID), lambda i: (i, 0)),
            pl.BlockSpec((R, 1), lambda i: (i, 0)),
            pl.BlockSpec((HID, HID), lambda i: (0, 0)),
            pl.BlockSpec((1, HID), lambda i: (0, 0)),
        ],
        out_specs=pl.BlockSpec((R, HID), lambda i: (i, 0)),
        out_shape=jax.ShapeDtypeStruct((N, HID), _F32),
    )(agg1, h1s, dinv, W2, b12)


def _final_tc(agg2, h2s, dinv, b22, batch2, Wl1, bl12, Wl2, bl22):
    def body(ag, h2, dv, b2, bt, wl1, bl1, wl2, bl2, out, acc, cnt):
        i = pl.program_id(0)
        a = ag[0] + ag[1] + h2[...]
        o2 = jnp.maximum(a * dv[...] + b2[...], 0.0)
        gids = lax.broadcasted_iota(jnp.int32, (R, G), 1)
        m = (bt[...] == gids).astype(_F32)
        psum = lax.dot_general(m, o2, (((0,), (0,)), ((), ())), preferred_element_type=_F32)
        pcnt = jnp.sum(m, axis=0)[:, None]

        @pl.when(i == 0)
        def _():
            acc[...] = jnp.zeros((G, HID), _F32)
            cnt[...] = jnp.zeros((G, HID), _F32)

        acc[...] = acc[...] + psum
        cnt[...] = cnt[...] + jnp.broadcast_to(pcnt, (G, HID))

        @pl.when(i == GRID - 1)
        def _():
            pooled = acc[...] / jnp.maximum(cnt[...], 1.0)
            z1 = jnp.maximum(jnp.dot(pooled, wl1[...], preferred_element_type=_F32) + bl1[...], 0.0)
            z = jnp.dot(z1, wl2[...], preferred_element_type=_F32) + bl2[...]
            zm = z - jnp.max(z, axis=-1, keepdims=True)
            out[...] = zm - jnp.log(jnp.sum(jnp.exp(zm), axis=-1, keepdims=True))

    return pl.pallas_call(
        body,
        grid=(GRID,),
        in_specs=[
            pl.BlockSpec((NC, R, HID), lambda i: (0, i, 0)),
            pl.BlockSpec((R, HID), lambda i: (i, 0)),
            pl.BlockSpec((R, 1), lambda i: (i, 0)),
            pl.BlockSpec((1, HID), lambda i: (0, 0)),
            pl.BlockSpec((R, 1), lambda i: (i, 0)),
            pl.BlockSpec((HID, HID), lambda i: (0, 0)),
            pl.BlockSpec((1, HID), lambda i: (0, 0)),
            pl.BlockSpec((HID, OUT), lambda i: (0, 0)),
            pl.BlockSpec((1, OUT), lambda i: (0, 0)),
        ],
        out_specs=pl.BlockSpec((G, OUT), lambda i: (0, 0)),
        out_shape=jax.ShapeDtypeStruct((G, OUT), _F32),
        scratch_shapes=[
            pltpu.VMEM((G, HID), _F32),
            pltpu.VMEM((G, HID), _F32),
        ],
    )(agg2, h2s, dinv, b22, batch2, Wl1, bl12, Wl2, bl22)


def kernel(content_x, bert_x, edge_index, batch, Wc, bc, Wb, bb,
           W1, b1, W2, b2, Wl1, bl1, Wl2, bl2):
    e = edge_index.shape[1]
    padn = E_PAD - e
    src = jnp.concatenate([edge_index[0], jnp.zeros((padn,), jnp.int32)])
    dst = jnp.concatenate([edge_index[1], jnp.full((padn,), DUMP, jnp.int32)])
    srcp = src.reshape(NS, KT, CHUNK)
    dstp = dst.reshape(NS, KT, CHUNK)

    zeros16 = jnp.zeros((RPT, 16), _F32)
    zeros32 = jnp.zeros((RPT, HID), _F32)
    ones16 = jnp.ones((CHUNK, 16), _F32)
    batch2 = batch.reshape(N, 1)
    bc2 = bc.reshape(1, HID)
    bb2 = bb.reshape(1, HID)
    b12 = b1.reshape(1, HID)
    b22 = b2.reshape(1, HID)
    bl12 = bl1.reshape(1, HID)
    bl22 = bl2.reshape(1, OUT)
    W1a = W1[:HID]
    W1b = W1[HID:]

    degp = _degree_sc(dstp, zeros16, ones16)
    h1s, dinv = _encoder_tc(content_x, bert_x, degp, Wc, bc2, Wb, bb2, W1a, W1b)
    agg1 = _aggregate_sc(srcp, dstp, h1s, zeros32)
    h2s = _mid_tc(agg1, h1s, dinv, W2, b12)
    agg2 = _aggregate_sc(srcp, dstp, h2s, zeros32)
    return _final_tc(agg2, h2s, dinv, b22, batch2, Wl1, bl12, Wl2, bl22)
